# Initial kernel scaffold; baseline (speedup 1.0000x reference)
import jax
import jax.numpy as jnp
from jax.experimental import pallas as pl


def kernel(x, edge_index, batch, W1, b1, W2, b2, bng, bnb, muW, mub, lvW, lvb, mug, mubb, lvg, lvbb):
    raise NotImplementedError("write your pallas kernel here")


# speedup vs baseline: 3.9623x; 3.9623x over previous
import functools

import jax
import jax.numpy as jnp
from jax import lax
from jax.experimental import pallas as pl
from jax.experimental.pallas import tpu as pltpu
from jax.experimental.pallas import tpu_sc as plsc

N = 10000
E = 320000
D = 128
L = 3

NC = 2
NS = 16
NW = NC * NS
CH = 128
NCHUNK = (E + NW * CH - 1) // (NW * CH)
EPAD = NW * CH * NCHUNK
NPAD = 10240
RPS = NPAD // NS

_DOT_PREC = lax.Precision.DEFAULT


def _segment_sum_sc(h, src_r, dst_r, zrows):
    mesh = plsc.VectorSubcoreMesh(core_axis_name="c", subcore_axis_name="s")

    @functools.partial(
        pl.kernel,
        mesh=mesh,
        out_type=jax.ShapeDtypeStruct((NC, NPAD, D), jnp.float32),
        scratch_types=[
            pltpu.VMEM((NCHUNK, CH), jnp.int32),
            pltpu.VMEM((NCHUNK, CH), jnp.int32),
            pltpu.VMEM((CH, D), jnp.float32),
            pltpu.VMEM_SHARED((NPAD, D), jnp.float32),
        ],
    )
    def segsum(h_hbm, src_hbm, dst_hbm, z_hbm, out_hbm, srcv, dstv, rows, acc):
        c = lax.axis_index("c")
        s = lax.axis_index("s")
        wid = s * NC + c
        pltpu.sync_copy(z_hbm, acc.at[pl.ds(s * RPS, RPS)])
        pltpu.sync_copy(src_hbm.at[wid], srcv)
        pltpu.sync_copy(dst_hbm.at[wid], dstv)
        plsc.subcore_barrier()

        @pl.loop(0, NCHUNK)
        def _(j):
            pltpu.sync_copy(h_hbm.at[srcv.at[j]], rows)
            pltpu.sync_copy(rows, acc.at[dstv.at[j]], add=True)

        plsc.subcore_barrier()
        pltpu.sync_copy(acc.at[pl.ds(s * RPS, RPS)],
                        out_hbm.at[c, pl.ds(s * RPS, RPS)])

    return segsum(h, src_r, dst_r, zrows)


_BLK = 1000
_GRID = N // _BLK


def _mlp_body(h_ref, agg_ref, w1_ref, b1_ref, w2_ref, b2_ref,
              a_ref, sum_ref, sq_ref):
    z = h_ref[...] + agg_ref[0] + agg_ref[1]
    y = lax.dot_general(z, w1_ref[...], (((1,), (0,)), ((), ())),
                        precision=_DOT_PREC,
                        preferred_element_type=jnp.float32) + b1_ref[...]
    y = jnp.maximum(y, 0.0)
    y = lax.dot_general(y, w2_ref[...], (((1,), (0,)), ((), ())),
                        precision=_DOT_PREC,
                        preferred_element_type=jnp.float32) + b2_ref[...]
    a = jnp.maximum(y, 0.0)
    a_ref[...] = a

    @pl.when(pl.program_id(0) == 0)
    def _():
        sum_ref[...] = jnp.zeros_like(sum_ref)
        sq_ref[...] = jnp.zeros_like(sq_ref)

    sum_ref[...] += jnp.sum(a, axis=0, keepdims=True)
    sq_ref[...] += jnp.sum(a * a, axis=0, keepdims=True)


def _mlp_layer(h, agg, w1, b1, w2, b2):
    out_shapes = (
        jax.ShapeDtypeStruct((N, D), jnp.float32),
        jax.ShapeDtypeStruct((1, D), jnp.float32),
        jax.ShapeDtypeStruct((1, D), jnp.float32),
    )
    full = lambda i: (0, 0)
    return pl.pallas_call(
        _mlp_body,
        grid=(_GRID,),
        in_specs=[
            pl.BlockSpec((_BLK, D), lambda i: (i, 0)),
            pl.BlockSpec((NC, _BLK, D), lambda i: (0, i, 0)),
            pl.BlockSpec((D, D), full),
            pl.BlockSpec((1, D), full),
            pl.BlockSpec((D, D), full),
            pl.BlockSpec((1, D), full),
        ],
        out_specs=(
            pl.BlockSpec((_BLK, D), lambda i: (i, 0)),
            pl.BlockSpec((1, D), full),
            pl.BlockSpec((1, D), full),
        ),
        out_shape=out_shapes,
    )(h, agg, w1, b1.reshape(1, D), w2, b2.reshape(1, D))


def _bn_body(a_ref, sum_ref, sq_ref, g_ref, b_ref, out_ref):
    inv_n = jnp.float32(1.0 / N)
    mean = sum_ref[...] * inv_n
    var = sq_ref[...] * inv_n - mean * mean
    scale = g_ref[...] * lax.rsqrt(var + 1e-5)
    out_ref[...] = a_ref[...] * scale + (b_ref[...] - mean * scale)


def _bn_apply(a, colsum, colsq, g, b):
    full = lambda i: (0, 0)
    return pl.pallas_call(
        _bn_body,
        grid=(_GRID,),
        in_specs=[
            pl.BlockSpec((_BLK, D), lambda i: (i, 0)),
            pl.BlockSpec((1, D), full),
            pl.BlockSpec((1, D), full),
            pl.BlockSpec((1, D), full),
            pl.BlockSpec((1, D), full),
        ],
        out_specs=pl.BlockSpec((_BLK, D), lambda i: (i, 0)),
        out_shape=jax.ShapeDtypeStruct((N, D), jnp.float32),
    )(a, colsum, colsq, g.reshape(1, D), b.reshape(1, D))


def _head_body(h_ref, muw_ref, mub_ref, lvw_ref, lvb_ref,
               amu_ref, alv_ref, smu_ref, qmu_ref, slv_ref, qlv_ref):
    h = h_ref[...]
    ymu = lax.dot_general(h, muw_ref[...], (((1,), (0,)), ((), ())),
                          precision=_DOT_PREC,
                          preferred_element_type=jnp.float32) + mub_ref[...]
    ymu = jnp.maximum(ymu, 0.0)
    ylv = lax.dot_general(h, lvw_ref[...], (((1,), (0,)), ((), ())),
                          precision=_DOT_PREC,
                          preferred_element_type=jnp.float32) + lvb_ref[...]
    ylv = jnp.maximum(ylv, 0.0)
    amu_ref[...] = ymu
    alv_ref[...] = ylv

    @pl.when(pl.program_id(0) == 0)
    def _():
        smu_ref[...] = jnp.zeros_like(smu_ref)
        qmu_ref[...] = jnp.zeros_like(qmu_ref)
        slv_ref[...] = jnp.zeros_like(slv_ref)
        qlv_ref[...] = jnp.zeros_like(qlv_ref)

    smu_ref[...] += jnp.sum(ymu, axis=0, keepdims=True)
    qmu_ref[...] += jnp.sum(ymu * ymu, axis=0, keepdims=True)
    slv_ref[...] += jnp.sum(ylv, axis=0, keepdims=True)
    qlv_ref[...] += jnp.sum(ylv * ylv, axis=0, keepdims=True)


def _heads(h, muW, mub, lvW, lvb):
    full = lambda i: (0, 0)
    row = pl.BlockSpec((_BLK, D), lambda i: (i, 0))
    stat = pl.BlockSpec((1, D), full)
    return pl.pallas_call(
        _head_body,
        grid=(_GRID,),
        in_specs=[row, pl.BlockSpec((D, D), full), stat,
                  pl.BlockSpec((D, D), full), stat],
        out_specs=(row, row, stat, stat, stat, stat),
        out_shape=(
            jax.ShapeDtypeStruct((N, D), jnp.float32),
            jax.ShapeDtypeStruct((N, D), jnp.float32),
            jax.ShapeDtypeStruct((1, D), jnp.float32),
            jax.ShapeDtypeStruct((1, D), jnp.float32),
            jax.ShapeDtypeStruct((1, D), jnp.float32),
            jax.ShapeDtypeStruct((1, D), jnp.float32),
        ),
    )(h, muW, mub.reshape(1, D), lvW, lvb.reshape(1, D))


def kernel(x, edge_index, batch, W1, b1, W2, b2, bng, bnb,
           muW, mub, lvW, lvb, mug, mubb, lvg, lvbb):
    pad = EPAD - E
    src = jnp.concatenate([edge_index[0], jnp.zeros((pad,), jnp.int32)])
    dst = jnp.concatenate([edge_index[1],
                           jnp.full((pad,), NPAD - 1, jnp.int32)])
    src_r = src.reshape(NW, NCHUNK, CH)
    dst_r = dst.reshape(NW, NCHUNK, CH)
    zrows = jnp.zeros((RPS, D), jnp.float32)

    h = x
    for i in range(L):
        agg = _segment_sum_sc(h, src_r, dst_r, zrows)
        a, csum, csq = _mlp_layer(h, agg, W1[i], b1[i], W2[i], b2[i])
        h = _bn_apply(a, csum, csq, bng[i], bnb[i])

    amu, alv, smu, qmu, slv, qlv = _heads(h, muW, mub, lvW, lvb)
    mu = _bn_apply(amu, smu, qmu, mug, mubb)
    lv = _bn_apply(alv, slv, qlv, lvg, lvbb)
    return (mu, lv)
